# baseline (speedup 1.0000x reference)
import functools

import numpy as np
import jax
import jax.numpy as jnp
from jax import lax
from jax.experimental import pallas as pl
from jax.experimental.pallas import tpu as pltpu
from jax.experimental.pallas import tpu_sc as plsc

N = 10000
E = 320000
F = 128
G = 64
NCLS = 40

NW = 32
K = 128
NCH = 80
EPW = NCH * K
EPAD = NW * EPW
NP = 10112
RPS = NP // 16

_f32 = jnp.float32


def _vgather(v, idx):
    dn = lax.GatherDimensionNumbers(
        offset_dims=(), collapsed_slice_dims=(0,), start_index_map=(0,))
    return lax.gather(v, idx[:, None], dn, (1,),
                      mode=lax.GatherScatterMode.PROMISE_IN_BOUNDS)


def _make_edge_pass(rw, wide):
    mesh = plsc.VectorSubcoreMesh(core_axis_name="c", subcore_axis_name="s")

    @functools.partial(
        pl.kernel,
        out_type=jax.ShapeDtypeStruct((2, NP, rw), _f32),
        mesh=mesh,
        compiler_params=pltpu.CompilerParams(use_tc_tiling_on_sc=False),
        scratch_types=[
            pltpu.VMEM((NCH, K), jnp.int32),
            pltpu.VMEM((NCH, K), jnp.int32),
            pltpu.VMEM((2, K, rw), _f32),
            pltpu.VMEM((2, K, 16), _f32),
            pltpu.VMEM((2, K, rw), _f32),
            pltpu.VMEM_SHARED((NP, rw), _f32),
            pltpu.SemaphoreType.DMA,
            pltpu.SemaphoreType.DMA,
            pltpu.SemaphoreType.DMA,
            pltpu.SemaphoreType.DMA,
        ],
    )
    def edge_pass(xcat_h, ald_h, src_h, dst_h, zr_h, out_h,
                  src_v, dst_v, xs_v, ad_v, y_v, acc,
                  sg0, sg1, ss0, ss1):
        ci = lax.axis_index("c")
        si = lax.axis_index("s")
        wid = si * 2 + ci
        sg = (sg0, sg1)
        ss = (ss0, ss1)
        pltpu.sync_copy(zr_h, acc.at[pl.ds(si * RPS, RPS)])
        pltpu.sync_copy(src_h.at[wid], src_v)
        pltpu.sync_copy(dst_h.at[wid], dst_v)
        plsc.subcore_barrier()

        idxb = (lax.iota(jnp.int32, 16) & 7) + 8
        mask_lo = lax.iota(jnp.int32, 16) < 8

        def g_start(j, b):
            pltpu.async_copy(xcat_h.at[src_v.at[j]], xs_v.at[b], sg[b])
            pltpu.async_copy(ald_h.at[dst_v.at[j]], ad_v.at[b], sg[b])

        def g_wait(b):
            pltpu.make_async_copy(xcat_h.at[src_v.at[0]], xs_v.at[b],
                                  sg[b]).wait()
            pltpu.make_async_copy(ald_h.at[dst_v.at[0]], ad_v.at[b],
                                  sg[b]).wait()

        def s_start(j, b):
            pltpu.async_copy(y_v.at[b], acc.at[dst_v.at[j]], ss[b],
                             add=True)

        def s_wait(b):
            pltpu.make_async_copy(y_v.at[b], acc.at[dst_v.at[0]],
                                  ss[b]).wait()

        U = 8

        def compute(b):
            xs_b = xs_v.at[b]
            ad_b = ad_v.at[b]
            y_b = y_v.at[b]
            if wide:
                def edge(q):
                    e0 = q * U
                    ts = []
                    for i in range(U):
                        s = xs_b[e0 + i, pl.ds(64, 16)] + ad_b[e0 + i, :]
                        ts.append(jnp.exp(jnp.maximum(s, s * 0.2)))
                    tbs = [_vgather(t, idxb) for t in ts]
                    for i in range(U):
                        y_b[e0 + i, pl.ds(64, 16)] = ts[i]
                    for i in range(U):
                        for v in range(4):
                            y_b[e0 + i, pl.ds(16 * v, 16)] = (
                                xs_b[e0 + i, pl.ds(16 * v, 16)] * tbs[i])
            else:
                def edge(q):
                    e0 = q * U
                    xss = [xs_b[e0 + i, :] for i in range(U)]
                    ts = []
                    for i in range(U):
                        s = xss[i] + ad_b[e0 + i, :]
                        ts.append(jnp.exp(jnp.maximum(s, s * 0.2)))
                    tbs = [_vgather(t, idxb) for t in ts]
                    for i in range(U):
                        y_b[e0 + i, :] = jnp.where(
                            mask_lo, xss[i] * tbs[i], tbs[i])
            plsc.parallel_loop(0, K // U, unroll=2)(edge)

        g_start(0, 0)

        def round_(r, carry):
            for b in range(2):
                j = 2 * r + b
                nb = 1 - b

                @pl.when(j + 1 < NCH)
                def _():
                    g_start(j + 1, nb)

                g_wait(b)

                @pl.when(j >= 2)
                def _():
                    s_wait(b)

                compute(b)
                s_start(j, b)
            return carry

        lax.fori_loop(0, NCH // 2, round_, 0)
        s_wait(0)
        s_wait(1)
        plsc.subcore_barrier()
        pltpu.sync_copy(acc.at[pl.ds(si * RPS, RPS)],
                        out_h.at[ci].at[pl.ds(si * RPS, RPS)])

    return edge_pass


_edge_wide = _make_edge_pass(80, True)
_edge_narrow = _make_edge_pass(16, False)


def _dot(a, b):
    return jnp.dot(a, b, preferred_element_type=_f32)


def _tc_enc_body(x_r, we_r, be_r, x0_o):
    x0_o[...] = _dot(x_r[...], we_r[...]) + be_r[...]


def _combine_bn_elu(p0, p1, b_r, g_r, be_r, hprev):
    num = p0[:N, :64] + p1[:N, :64]
    den = jnp.tile(p0[:N, 72:] + p1[:N, 72:], (1, 8))
    gat = num / (den + 1e-16) + b_r[...]
    mu = jnp.mean(gat, axis=0, keepdims=True)
    var = jnp.mean((gat - mu) ** 2, axis=0, keepdims=True)
    z = g_r[...] * (gat - mu) * lax.rsqrt(var + 1e-5) + be_r[...]
    return hprev + jnp.where(z > 0, z, jnp.exp(z) - 1.0)


def _tc_combine_body(p0_r, p1_r, hprev_r, b_r, g_r, be_r, xn_o):
    xn_o[...] = _combine_bn_elu(p0_r[...], p1_r[...], b_r, g_r, be_r,
                                hprev_r[...])


def _tc_prep_body(xn_r, wn_r, pm_r, xcat_o, ald_o):
    xw = _dot(xn_r[...], wn_r[...])
    ald = _dot(xw, pm_r[...])
    zp = jnp.zeros((NP - N, 80), _f32)
    xcat_o[...] = jnp.concatenate(
        [jnp.concatenate([xw, jnp.zeros((N, 8), _f32), ald[:, :8]], axis=1),
         zp], axis=0)
    ald_o[...] = jnp.concatenate(
        [jnp.concatenate([jnp.zeros((N, 8), _f32), ald[:, 8:]], axis=1),
         zp[:, :16]], axis=0)


def _tc_prep3_body(xn_r, w3_r, a3_r, xcat_o, ald_o):
    xw3 = _dot(xn_r[...], w3_r[...])
    a3 = _dot(xw3, a3_r[...])
    ones8 = jnp.ones((1, 8), _f32)
    zp = jnp.zeros((NP - N, 16), _f32)
    xcat_o[...] = jnp.concatenate(
        [jnp.concatenate([xw3, a3[:, :1] * ones8], axis=1), zp], axis=0)
    ald_o[...] = jnp.concatenate(
        [jnp.concatenate([jnp.zeros((N, 8), _f32), a3[:, 1:] * ones8],
                         axis=1), zp], axis=0)


def _tc_final_body(p0_r, p1_r, bt_r, b3_r, g3_r, be3_r, wl_r, bl_r, out_o):
    p0 = p0_r[...]
    p1 = p1_r[...]
    num = p0[:N, :8] + p1[:N, :8]
    den = p0[:N, 8:9] + p1[:N, 8:9]
    x3 = num / (den + 1e-16) + b3_r[...]
    mu = jnp.mean(x3, axis=0, keepdims=True)
    var = jnp.mean((x3 - mu) ** 2, axis=0, keepdims=True)
    x3 = g3_r[...] * (x3 - mu) * lax.rsqrt(var + 1e-5) + be3_r[...]
    oh = (lax.broadcasted_iota(jnp.int32, (G, N), 0) == bt_r[...]).astype(_f32)
    cnt = jnp.sum(oh, axis=1, keepdims=True)
    pooled = _dot(oh, x3) / jnp.maximum(cnt, 1.0)
    out_o[...] = _dot(pooled, wl_r[...]) + bl_r[...]


def _call(body, out_shapes, *args):
    return pl.pallas_call(
        body,
        out_shape=[jax.ShapeDtypeStruct(s, _f32) for s in out_shapes],
        compiler_params=pltpu.CompilerParams(
            vmem_limit_bytes=128 * 1024 * 1024),
    )(*args)


_w = np.arange(64)
_PERM = (_w % 8) * 8 + _w // 8
_H = _w % 8
_C = _w // 8
_OH8 = np.eye(8, dtype=np.float32)[_H]


def _pmat(a_s, a_d):
    oh = jnp.asarray(_OH8)
    vs = a_s[_H, _C]
    vd = a_d[_H, _C]
    return jnp.concatenate([oh * vs[:, None], oh * vd[:, None]], axis=1)


def kernel(x, edge_index, batch, W_enc, b_enc, W1, as1, ad1, b1, g1, be1,
           W2, as2, ad2, b2, g2, be2, W3, as3, ad3, b3, g3, be3,
           W_lin, b_lin):
    p = _PERM
    ppw = EPW - E // NW
    pad_src = jnp.full((NW, ppw), N, jnp.int32)
    pad_dst = jnp.broadcast_to(
        N + (jnp.arange(ppw, dtype=jnp.int32) % (NP - N)), (NW, ppw))
    src = jnp.concatenate(
        [edge_index[0].reshape(NW, E // NW), pad_src], axis=1).reshape(
            NW, NCH, K)
    dst = jnp.concatenate(
        [edge_index[1].reshape(NW, E // NW), pad_dst], axis=1).reshape(
            NW, NCH, K)
    zr80 = jnp.zeros((RPS, 80), _f32)
    zr16 = jnp.zeros((RPS, 16), _f32)
    bt = batch.reshape(1, N)

    Wenc_p = W_enc[:, p]
    W1pp = W1[p][:, p]
    W2pp = W2[p][:, p]
    W3p = W3[p, :]
    Pm1 = _pmat(as1, ad1)
    Pm2 = _pmat(as2, ad2)
    a3 = jnp.concatenate([as3.reshape(8, 1), ad3.reshape(8, 1)], axis=1)

    r1 = lambda v: v.reshape(1, -1)

    (x0,) = _call(_tc_enc_body, [(N, 64)], x, Wenc_p, r1(b_enc[p]))
    xcat1, ald1 = _call(_tc_prep_body, [(NP, 80), (NP, 16)], x0, W1pp, Pm1)

    parts1 = _edge_wide(xcat1, ald1, src, dst, zr80)
    (x1,) = _call(
        _tc_combine_body, [(N, 64)],
        parts1[0], parts1[1], x0, r1(b1[p]), r1(g1[p]), r1(be1[p]))
    xcat2, ald2 = _call(_tc_prep_body, [(NP, 80), (NP, 16)], x1, W2pp, Pm2)

    parts2 = _edge_wide(xcat2, ald2, src, dst, zr80)
    (x2,) = _call(
        _tc_combine_body, [(N, 64)],
        parts2[0], parts2[1], x1, r1(b2[p]), r1(g2[p]), r1(be2[p]))
    xcat3, ald3 = _call(_tc_prep3_body, [(NP, 16), (NP, 16)], x2, W3p, a3)

    parts3 = _edge_narrow(xcat3, ald3, src, dst, zr16)
    (out,) = _call(
        _tc_final_body, [(G, NCLS)],
        parts3[0], parts3[1], bt, r1(b3), r1(g3), r1(be3), W_lin, r1(b_lin))
    return out


# speedup vs baseline: 1.0534x; 1.0534x over previous
import functools

import numpy as np
import jax
import jax.numpy as jnp
from jax import lax
from jax.experimental import pallas as pl
from jax.experimental.pallas import tpu as pltpu
from jax.experimental.pallas import tpu_sc as plsc

N = 10000
E = 320000
F = 128
G = 64
NCLS = 40

NW = 32
K = 128
NCH = 80
EPW = NCH * K
EPAD = NW * EPW
NP = 10112
RPS = NP // 16

_f32 = jnp.float32


def _vgather(v, idx):
    dn = lax.GatherDimensionNumbers(
        offset_dims=(), collapsed_slice_dims=(0,), start_index_map=(0,))
    return lax.gather(v, idx[:, None], dn, (1,),
                      mode=lax.GatherScatterMode.PROMISE_IN_BOUNDS)


def _make_edge_pass(rw, wide):
    mesh = plsc.VectorSubcoreMesh(core_axis_name="c", subcore_axis_name="s")

    @functools.partial(
        pl.kernel,
        out_type=jax.ShapeDtypeStruct((2, NP, rw), _f32),
        mesh=mesh,
        compiler_params=pltpu.CompilerParams(use_tc_tiling_on_sc=False),
        scratch_types=[
            pltpu.VMEM((NCH, K), jnp.int32),
            pltpu.VMEM((NCH, K), jnp.int32),
            pltpu.VMEM((2, K, rw), _f32),
            pltpu.VMEM((2, K, 16), _f32),
            pltpu.VMEM((2, K, rw), _f32),
            pltpu.VMEM_SHARED((NP, rw), _f32),
            pltpu.SemaphoreType.DMA,
            pltpu.SemaphoreType.DMA,
            pltpu.SemaphoreType.DMA,
            pltpu.SemaphoreType.DMA,
        ],
    )
    def edge_pass(xcat_h, ald_h, src_h, dst_h, zr_h, out_h,
                  src_v, dst_v, xs_v, ad_v, y_v, acc,
                  sg0, sg1, ss0, ss1):
        ci = lax.axis_index("c")
        si = lax.axis_index("s")
        wid = si * 2 + ci
        sg = (sg0, sg1)
        ss = (ss0, ss1)
        pltpu.sync_copy(zr_h, acc.at[pl.ds(si * RPS, RPS)])
        pltpu.sync_copy(src_h.at[wid], src_v)
        pltpu.sync_copy(dst_h.at[wid], dst_v)
        plsc.subcore_barrier()

        idxb = (lax.iota(jnp.int32, 16) & 7) + 8
        mask_lo = lax.iota(jnp.int32, 16) < 8

        def g_start(j, b):
            pltpu.async_copy(xcat_h.at[src_v.at[j]], xs_v.at[b], sg[b])
            pltpu.async_copy(ald_h.at[dst_v.at[j]], ad_v.at[b], sg[b])

        def g_wait(b):
            pltpu.make_async_copy(xcat_h.at[src_v.at[0]], xs_v.at[b],
                                  sg[b]).wait()
            pltpu.make_async_copy(ald_h.at[dst_v.at[0]], ad_v.at[b],
                                  sg[b]).wait()

        def s_start(j, b):
            pltpu.async_copy(y_v.at[b], acc.at[dst_v.at[j]], ss[b],
                             add=True)

        def s_wait(b):
            pltpu.make_async_copy(y_v.at[b], acc.at[dst_v.at[0]],
                                  ss[b]).wait()

        U = 16

        def compute(b):
            xs_b = xs_v.at[b]
            ad_b = ad_v.at[b]
            y_b = y_v.at[b]
            if wide:
                def edge(q):
                    e0 = q * U
                    ts = []
                    for i in range(U):
                        s = xs_b[e0 + i, pl.ds(64, 16)] + ad_b[e0 + i, :]
                        ts.append(jnp.exp(jnp.maximum(s, s * 0.2)))
                    tbs = [_vgather(t, idxb) for t in ts]
                    for i in range(U):
                        y_b[e0 + i, pl.ds(64, 16)] = ts[i]
                    for i in range(U):
                        for v in range(4):
                            y_b[e0 + i, pl.ds(16 * v, 16)] = (
                                xs_b[e0 + i, pl.ds(16 * v, 16)] * tbs[i])
            else:
                def edge(q):
                    e0 = q * U
                    xss = [xs_b[e0 + i, :] for i in range(U)]
                    ts = []
                    for i in range(U):
                        s = xss[i] + ad_b[e0 + i, :]
                        ts.append(jnp.exp(jnp.maximum(s, s * 0.2)))
                    tbs = [_vgather(t, idxb) for t in ts]
                    for i in range(U):
                        y_b[e0 + i, :] = jnp.where(
                            mask_lo, xss[i] * tbs[i], tbs[i])
            plsc.parallel_loop(0, K // U, unroll=1)(edge)

        g_start(0, 0)

        def round_(r, carry):
            for b in range(2):
                j = 2 * r + b
                nb = 1 - b

                @pl.when(j + 1 < NCH)
                def _():
                    g_start(j + 1, nb)

                g_wait(b)

                @pl.when(j >= 2)
                def _():
                    s_wait(b)

                compute(b)
                s_start(j, b)
            return carry

        lax.fori_loop(0, NCH // 2, round_, 0)
        s_wait(0)
        s_wait(1)
        plsc.subcore_barrier()
        pltpu.sync_copy(acc.at[pl.ds(si * RPS, RPS)],
                        out_h.at[ci].at[pl.ds(si * RPS, RPS)])

    return edge_pass


_edge_wide = _make_edge_pass(80, True)
_edge_narrow = _make_edge_pass(16, False)


def _dot(a, b):
    return jnp.dot(a, b, preferred_element_type=_f32)


def _tc_enc_body(x_r, we_r, be_r, x0_o):
    x0_o[...] = _dot(x_r[...], we_r[...]) + be_r[...]


def _combine_bn_elu(p0, p1, b_r, g_r, be_r, hprev):
    num = p0[:N, :64] + p1[:N, :64]
    den = jnp.tile(p0[:N, 72:] + p1[:N, 72:], (1, 8))
    gat = num / (den + 1e-16) + b_r[...]
    mu = jnp.mean(gat, axis=0, keepdims=True)
    var = jnp.mean((gat - mu) ** 2, axis=0, keepdims=True)
    z = g_r[...] * (gat - mu) * lax.rsqrt(var + 1e-5) + be_r[...]
    return hprev + jnp.where(z > 0, z, jnp.exp(z) - 1.0)


def _tc_combine_body(p0_r, p1_r, hprev_r, b_r, g_r, be_r, xn_o):
    xn_o[...] = _combine_bn_elu(p0_r[...], p1_r[...], b_r, g_r, be_r,
                                hprev_r[...])


def _tc_prep_body(xn_r, wn_r, pm_r, xcat_o, ald_o):
    xw = _dot(xn_r[...], wn_r[...])
    ald = _dot(xw, pm_r[...])
    zp = jnp.zeros((NP - N, 80), _f32)
    xcat_o[...] = jnp.concatenate(
        [jnp.concatenate([xw, jnp.zeros((N, 8), _f32), ald[:, :8]], axis=1),
         zp], axis=0)
    ald_o[...] = jnp.concatenate(
        [jnp.concatenate([jnp.zeros((N, 8), _f32), ald[:, 8:]], axis=1),
         zp[:, :16]], axis=0)


def _tc_prep3_body(xn_r, w3_r, a3_r, xcat_o, ald_o):
    xw3 = _dot(xn_r[...], w3_r[...])
    a3 = _dot(xw3, a3_r[...])
    ones8 = jnp.ones((1, 8), _f32)
    zp = jnp.zeros((NP - N, 16), _f32)
    xcat_o[...] = jnp.concatenate(
        [jnp.concatenate([xw3, a3[:, :1] * ones8], axis=1), zp], axis=0)
    ald_o[...] = jnp.concatenate(
        [jnp.concatenate([jnp.zeros((N, 8), _f32), a3[:, 1:] * ones8],
                         axis=1), zp], axis=0)


def _tc_final_body(p0_r, p1_r, bt_r, b3_r, g3_r, be3_r, wl_r, bl_r, out_o):
    p0 = p0_r[...]
    p1 = p1_r[...]
    num = p0[:N, :8] + p1[:N, :8]
    den = p0[:N, 8:9] + p1[:N, 8:9]
    x3 = num / (den + 1e-16) + b3_r[...]
    mu = jnp.mean(x3, axis=0, keepdims=True)
    var = jnp.mean((x3 - mu) ** 2, axis=0, keepdims=True)
    x3 = g3_r[...] * (x3 - mu) * lax.rsqrt(var + 1e-5) + be3_r[...]
    oh = (lax.broadcasted_iota(jnp.int32, (G, N), 0) == bt_r[...]).astype(_f32)
    cnt = jnp.sum(oh, axis=1, keepdims=True)
    pooled = _dot(oh, x3) / jnp.maximum(cnt, 1.0)
    out_o[...] = _dot(pooled, wl_r[...]) + bl_r[...]


def _call(body, out_shapes, *args):
    return pl.pallas_call(
        body,
        out_shape=[jax.ShapeDtypeStruct(s, _f32) for s in out_shapes],
        compiler_params=pltpu.CompilerParams(
            vmem_limit_bytes=128 * 1024 * 1024),
    )(*args)


_w = np.arange(64)
_PERM = (_w % 8) * 8 + _w // 8
_H = _w % 8
_C = _w // 8
_OH8 = np.eye(8, dtype=np.float32)[_H]


def _pmat(a_s, a_d):
    oh = jnp.asarray(_OH8)
    vs = a_s[_H, _C]
    vd = a_d[_H, _C]
    return jnp.concatenate([oh * vs[:, None], oh * vd[:, None]], axis=1)


def kernel(x, edge_index, batch, W_enc, b_enc, W1, as1, ad1, b1, g1, be1,
           W2, as2, ad2, b2, g2, be2, W3, as3, ad3, b3, g3, be3,
           W_lin, b_lin):
    p = _PERM
    ppw = EPW - E // NW
    pad_src = jnp.full((NW, ppw), N, jnp.int32)
    pad_dst = jnp.broadcast_to(
        N + (jnp.arange(ppw, dtype=jnp.int32) % (NP - N)), (NW, ppw))
    src = jnp.concatenate(
        [edge_index[0].reshape(NW, E // NW), pad_src], axis=1).reshape(
            NW, NCH, K)
    dst = jnp.concatenate(
        [edge_index[1].reshape(NW, E // NW), pad_dst], axis=1).reshape(
            NW, NCH, K)
    zr80 = jnp.zeros((RPS, 80), _f32)
    zr16 = jnp.zeros((RPS, 16), _f32)
    bt = batch.reshape(1, N)

    Wenc_p = W_enc[:, p]
    W1pp = W1[p][:, p]
    W2pp = W2[p][:, p]
    W3p = W3[p, :]
    Pm1 = _pmat(as1, ad1)
    Pm2 = _pmat(as2, ad2)
    a3 = jnp.concatenate([as3.reshape(8, 1), ad3.reshape(8, 1)], axis=1)

    r1 = lambda v: v.reshape(1, -1)

    (x0,) = _call(_tc_enc_body, [(N, 64)], x, Wenc_p, r1(b_enc[p]))
    xcat1, ald1 = _call(_tc_prep_body, [(NP, 80), (NP, 16)], x0, W1pp, Pm1)

    parts1 = _edge_wide(xcat1, ald1, src, dst, zr80)
    (x1,) = _call(
        _tc_combine_body, [(N, 64)],
        parts1[0], parts1[1], x0, r1(b1[p]), r1(g1[p]), r1(be1[p]))
    xcat2, ald2 = _call(_tc_prep_body, [(NP, 80), (NP, 16)], x1, W2pp, Pm2)

    parts2 = _edge_wide(xcat2, ald2, src, dst, zr80)
    (x2,) = _call(
        _tc_combine_body, [(N, 64)],
        parts2[0], parts2[1], x1, r1(b2[p]), r1(g2[p]), r1(be2[p]))
    xcat3, ald3 = _call(_tc_prep3_body, [(NP, 16), (NP, 16)], x2, W3p, a3)

    parts3 = _edge_narrow(xcat3, ald3, src, dst, zr16)
    (out,) = _call(
        _tc_final_body, [(G, NCLS)],
        parts3[0], parts3[1], bt, r1(b3), r1(g3), r1(be3), W_lin, r1(b_lin))
    return out
